# Initial kernel scaffold; baseline (speedup 1.0000x reference)
import jax
import jax.numpy as jnp
from jax.experimental import pallas as pl


def kernel(out1, out, shorty, y_inds, parent):
    raise NotImplementedError("write your pallas kernel here")


# speedup vs baseline: 12.7083x; 12.7083x over previous
import functools

import jax
import jax.numpy as jnp
from jax import lax
from jax.experimental import pallas as pl
from jax.experimental.pallas import tpu as pltpu
from jax.experimental.pallas import tpu_sc as plsc

_NUMY = 100000
_NUM_CLUSTERS = 8192
_GAMMA = 1.0
_LANES = 16
_ROWS_BLK = 128


def _sc_gather_body(n_idx, parent_hbm, yidx_hbm, cidx_hbm, tbl_v, idx_v, res_v):
    cid = lax.axis_index("c")
    sid = lax.axis_index("s")

    @pl.when(jnp.logical_and(cid == 0, sid == 0))
    def _():
        pltpu.sync_copy(parent_hbm, tbl_v)
        pltpu.sync_copy(yidx_hbm, idx_v)

        def body(i, carry):
            iv = idx_v[pl.ds(i * _LANES, _LANES)]
            res_v[pl.ds(i * _LANES, _LANES)] = plsc.load_gather(tbl_v, [iv])
            return carry

        lax.fori_loop(0, n_idx // _LANES, body, 0)
        pltpu.sync_copy(res_v, cidx_hbm)


def _sc_parent_gather(parent_padded, yidx_flat):
    n_idx = yidx_flat.shape[0]
    tbl_n = parent_padded.shape[0]
    return pl.kernel(
        functools.partial(_sc_gather_body, n_idx),
        out_type=jax.ShapeDtypeStruct((n_idx,), jnp.int32),
        mesh=plsc.VectorSubcoreMesh(core_axis_name="c", subcore_axis_name="s"),
        compiler_params=pltpu.CompilerParams(needs_layout_passes=False),
        scratch_types=[
            pltpu.VMEM((tbl_n,), jnp.int32),
            pltpu.VMEM((n_idx,), jnp.int32),
            pltpu.VMEM((n_idx,), jnp.int32),
        ],
    )(parent_padded, yidx_flat)


def _bce_body(inv0, inv1, out1_ref, out_ref, shorty_ref, yinds_ref, cidx_ref,
              acc_ref):
    i = pl.program_id(0)

    x = out_ref[...]
    sh = shorty_ref[...]
    yi = yinds_ref[...]
    m = sh == yi[:, 0:1]
    for k in range(1, yi.shape[1]):
        m = jnp.logical_or(m, sh == yi[:, k:k + 1])
    m = jnp.logical_and(m, sh != _NUMY)
    s0 = jnp.sum(jnp.maximum(x, 0.0) + jnp.log1p(jnp.exp(-jnp.abs(x)))
                 - jnp.where(m, x, 0.0))

    x1 = out1_ref[...]
    col = lax.broadcasted_iota(jnp.int32, x1.shape, 1)
    ci = cidx_ref[...]
    m1 = col == ci[:, 0:1]
    for k in range(1, ci.shape[1]):
        m1 = jnp.logical_or(m1, col == ci[:, k:k + 1])
    m1 = jnp.logical_and(m1, col != _NUM_CLUSTERS)
    s1 = jnp.sum(jnp.maximum(x1, 0.0) + jnp.log1p(jnp.exp(-jnp.abs(x1)))
                 - jnp.where(m1, x1, 0.0))

    part = s0 * inv0 + (_GAMMA * inv1) * s1

    @pl.when(i == 0)
    def _():
        acc_ref[...] = jnp.zeros_like(acc_ref)

    acc_ref[...] += jnp.reshape(part, (1, 1))


def _bce_pallas(out1, out, shorty, y_inds, cidx, interpret=False):
    b, beam = out.shape
    ncp1 = out1.shape[1]
    lp = y_inds.shape[1]
    nblk = b // _ROWS_BLK
    inv0 = 1.0 / (b * beam)
    inv1 = 1.0 / (b * ncp1)
    acc = pl.pallas_call(
        functools.partial(_bce_body, inv0, inv1),
        grid=(nblk,),
        in_specs=[
            pl.BlockSpec((_ROWS_BLK, ncp1), lambda i: (i, 0)),
            pl.BlockSpec((_ROWS_BLK, beam), lambda i: (i, 0)),
            pl.BlockSpec((_ROWS_BLK, beam), lambda i: (i, 0)),
            pl.BlockSpec((_ROWS_BLK, lp), lambda i: (i, 0)),
            pl.BlockSpec((_ROWS_BLK, lp), lambda i: (i, 0)),
        ],
        out_specs=pl.BlockSpec((1, 1), lambda i: (0, 0)),
        out_shape=jax.ShapeDtypeStruct((1, 1), jnp.float32),
        interpret=interpret,
    )(out1, out, shorty, y_inds, cidx)
    return acc[0, 0]


def kernel(out1, out, shorty, y_inds, parent):
    b, lp = y_inds.shape
    tbl_n = (parent.shape[0] + _LANES - 1) // _LANES * _LANES
    parent_padded = jnp.pad(parent, (0, tbl_n - parent.shape[0]))
    cidx = _sc_parent_gather(parent_padded, y_inds.reshape(-1)).reshape(b, lp)
    return _bce_pallas(out1, out, shorty, y_inds, cidx)
